# baseline (speedup 1.0000x reference)
import functools

import jax
import jax.numpy as jnp
from jax import lax
from jax.experimental import pallas as pl
from jax.experimental.pallas import tpu as pltpu
from jax.experimental.pallas import tpu_sc as plsc

_PS = 7
_STRIDE = 2
_NPROJ = 256
_D = 147
_DP = 160
_L = 3721
_LP = 3840
_CHUNK = _LP // 16
_GCH = 128
_NCH = _LP // _GCH
_B = 4
_NTASK = _B * _NPROJ
_NTILE = 32
_TPT = _NTASK // _NTILE


def _patches_t(img):
    p = lax.conv_general_dilated_patches(
        img, filter_shape=(_PS, _PS), window_strides=(_STRIDE, _STRIDE),
        padding="VALID")
    return p.reshape(img.shape[0], _D, _L)


def _proj_tc_kernel(randT_ref, xT_ref, out_ref):
    r = randT_ref[...]
    mu = jnp.mean(r, axis=1, keepdims=True)
    var = jnp.sum((r - mu) ** 2, axis=1, keepdims=True) * (1.0 / (_D - 1))
    rn = r * lax.rsqrt(var)
    x = xT_ref[0]
    acc = lax.dot_general(rn, x, (((1,), (0,)), ((), ())),
                          preferred_element_type=jnp.float32)
    b = lax.bitcast_convert_type(acc, jnp.int32)
    mono = jnp.where(acc < 0, ~b, b | jnp.int32(-2147483648))
    col = lax.broadcasted_iota(jnp.int32, mono.shape, 1)
    out_ref[0] = jnp.where(col >= _L, jnp.int32(-1), mono)


_sc_mesh = plsc.VectorSubcoreMesh(core_axis_name="c", subcore_axis_name="s")


@functools.partial(
    pl.kernel,
    mesh=_sc_mesh,
    compiler_params=pltpu.CompilerParams(
        needs_layout_passes=False, use_tc_tiling_on_sc=False),
    out_type=jax.ShapeDtypeStruct((_NTASK * 16,), jnp.float32),
    scratch_types=[
        pltpu.VMEM((_LP,), jnp.int32),
        pltpu.VMEM((_LP,), jnp.int32),
        pltpu.VMEM((_LP,), jnp.int32),
        pltpu.VMEM((_LP,), jnp.int32),
        pltpu.VMEM((_LP,), jnp.int32),
        pltpu.VMEM((_LP,), jnp.int32),
        pltpu.VMEM((4096,), jnp.int32),
        pltpu.VMEM((_GCH, _DP), jnp.bfloat16),
        pltpu.VMEM((_GCH, _DP), jnp.bfloat16),
        pltpu.VMEM((_GCH, _DP), jnp.bfloat16),
        pltpu.VMEM((_TPT * 16,), jnp.float32),
        pltpu.SemaphoreType.DMA,
        pltpu.SemaphoreType.DMA,
        pltpu.SemaphoreType.DMA,
        pltpu.SemaphoreType.DMA,
        pltpu.SemaphoreType.DMA,
        pltpu.SemaphoreType.DMA,
    ],
)
def _sc_swd_kernel(keys_hbm, xp_hbm, yn_hbm, out_hbm,
                   kA, kB, pA, pB, idxX, idxY, hist,
                   b0, b1, b2, res,
                   sx0, sx1, sx2, sy0, sy1, sy2):
    wid = lax.axis_index("s") * 2 + lax.axis_index("c")
    lanes = lax.iota(jnp.int32, 16)
    ones = jnp.ones((16,), jnp.int32)
    zeros16 = jnp.zeros((16,), jnp.int32)
    gb0 = lanes * _CHUNK

    def radix_pass(kin, pin, kout, pout, shift, first, last, pbase):
        def zb(i, c):
            hist[pl.ds(i * 16, 16)] = zeros16
            return c
        lax.fori_loop(0, 256, zb, 0)

        def ph1(t, c):
            k = plsc.load_gather(kin, [gb0 + t])
            dg = lax.shift_right_logical(k, shift) & 255
            plsc.addupdate_scatter(hist, [dg * 16 + lanes], ones)
            return c
        lax.fori_loop(0, _CHUNK, ph1, 0)

        def ph2(i, carry):
            v = hist[pl.ds(i * 16, 16)]
            inc = plsc.cumsum(v)
            hist[pl.ds(i * 16, 16)] = inc - v + carry
            return carry + jnp.sum(v)
        lax.fori_loop(0, 256, ph2, jnp.int32(0))

        def ph3(t, c):
            gidx = gb0 + t
            k = plsc.load_gather(kin, [gidx])
            dg = lax.shift_right_logical(k, shift) & 255
            addr = dg * 16 + lanes
            off = plsc.load_gather(hist, [addr])
            if first:
                p = gidx + pbase
            else:
                p = plsc.load_gather(pin, [gidx])
            if not last:
                plsc.store_scatter(kout, [off], k)
            plsc.store_scatter(pout, [off], p)
            plsc.addupdate_scatter(hist, [addr], ones)
            return c
        lax.fori_loop(0, _CHUNK, ph3, 0)

    def sort_side(row, idx_out, pbase):
        pltpu.sync_copy(keys_hbm.at[row], kA)
        radix_pass(kA, None, kB, pB, 0, True, False, pbase)
        radix_pass(kB, pB, kA, pA, 8, False, False, pbase)
        radix_pass(kA, pA, kB, pB, 16, False, False, pbase)
        radix_pass(kB, pB, kA, idx_out, 24, False, True, pbase)

    def task_body(r, c):
        task = wid * _TPT + r
        s = task // _NPROJ
        j = task - s * _NPROJ
        pbase = s * _LP
        sort_side((s * 2) * _NPROJ + j, idxX, pbase)
        sort_side((s * 2 + 1) * _NPROJ + j, idxY, pbase)

        bufs = ((b0, sx0, sy0), (b1, sx1, sy1), (b2, sx2, sy2))

        def issue_x(ci, slot):
            buf, sx, _ = bufs[slot]
            pltpu.async_copy(xp_hbm.at[idxX.at[pl.ds(ci * _GCH, _GCH)]],
                             buf, sx)

        def issue_yadd(ci, slot):
            buf, _, sy = bufs[slot]
            pltpu.async_copy(yn_hbm.at[idxY.at[pl.ds(ci * _GCH, _GCH)]],
                             buf, sy, add=True)

        def wait_x(slot):
            buf, sx, _ = bufs[slot]
            pltpu.make_async_copy(xp_hbm.at[pl.ds(0, _GCH)], buf, sx).wait()

        def wait_y(slot):
            buf, _, sy = bufs[slot]
            pltpu.make_async_copy(xp_hbm.at[pl.ds(0, _GCH)], buf, sy).wait()

        issue_x(0, 0)
        issue_x(1, 1)
        wait_x(0)
        issue_yadd(0, 0)

        def chunk_trip(i, acc):
            for b in (0, 1, 2):
                c = 3 * i + b
                m1 = (b + 1) % 3

                @pl.when(c + 1 < _NCH)
                def _():
                    wait_x(m1)
                    issue_yadd(c + 1, m1)

                buf = bufs[b][0]
                wait_y(b)

                def rowloop(rr, a):
                    d = buf[rr, pl.ds(0, 32)]
                    rs = jnp.abs(d)
                    for q in range(1, _DP // 32):
                        rs = rs + jnp.abs(buf[rr, pl.ds(q * 32, 32)])
                    lo, hi = plsc.unpack(
                        rs, format=plsc.PackFormat.INTERLEAVED)
                    return a + lo + hi
                acc = lax.fori_loop(0, _GCH, rowloop, acc)

                @pl.when(c + 2 < _NCH)
                def _():
                    issue_x(c + 2, (b + 2) % 3)
            return acc

        acc = lax.fori_loop(0, _NCH // 3, chunk_trip,
                            jnp.zeros((16,), jnp.float32))
        res[pl.ds(r * 16, 16)] = acc
        return c

    lax.fori_loop(0, _TPT, task_body, 0)
    pltpu.sync_copy(res, out_hbm.at[pl.ds(wid * _TPT * 16, _TPT * 16)])


def kernel(x, y, rand):
    xT = _patches_t(x)
    yT = _patches_t(y)
    xTp = jnp.pad(xT, ((0, 0), (0, 0), (0, _LP - _L)))
    yTp = jnp.pad(yT, ((0, 0), (0, 0), (0, _LP - _L)))
    xyT = jnp.stack([xTp, yTp], axis=1).reshape(2 * _B, _D, _LP)
    randT = jnp.transpose(rand)

    keys = pl.pallas_call(
        _proj_tc_kernel,
        grid=(2 * _B,),
        in_specs=[
            pl.BlockSpec((_NPROJ, _D), lambda i: (0, 0)),
            pl.BlockSpec((1, _D, _LP), lambda i: (i, 0, 0)),
        ],
        out_specs=pl.BlockSpec((1, _NPROJ, _LP), lambda i: (i, 0, 0)),
        out_shape=jax.ShapeDtypeStruct((2 * _B, _NPROJ, _LP), jnp.int32),
    )(randT, xyT)
    keys2 = keys.reshape(2 * _B * _NPROJ, _LP)

    xp = jnp.pad(jnp.transpose(xT, (0, 2, 1)),
                 ((0, 0), (0, _LP - _L), (0, _DP - _D))
                 ).reshape(_B * _LP, _DP).astype(jnp.bfloat16)
    yn = jnp.pad(jnp.transpose(-yT, (0, 2, 1)),
                 ((0, 0), (0, _LP - _L), (0, _DP - _D))
                 ).reshape(_B * _LP, _DP).astype(jnp.bfloat16)

    sums = _sc_swd_kernel(keys2, xp, yn)
    per_sample = sums.reshape(_B, _NPROJ * 16).sum(axis=1)
    return jnp.mean(per_sample / jnp.float32(_L * _D * _NPROJ))


# speedup vs baseline: 1.4313x; 1.4313x over previous
import functools

import jax
import jax.numpy as jnp
from jax import lax
from jax.experimental import pallas as pl
from jax.experimental.pallas import tpu as pltpu
from jax.experimental.pallas import tpu_sc as plsc

_PS = 7
_STRIDE = 2
_NPROJ = 256
_D = 147
_DP = 160
_L = 3721
_LP = 3840
_CHUNK = _LP // 16
_GCH = 128
_NCH = _LP // _GCH
_B = 4
_NTASK = _B * _NPROJ
_NTILE = 32
_TPT = _NTASK // _NTILE


def _patches_t(img):
    p = lax.conv_general_dilated_patches(
        img, filter_shape=(_PS, _PS), window_strides=(_STRIDE, _STRIDE),
        padding="VALID")
    return p.reshape(img.shape[0], _D, _L)


def _proj_tc_kernel(randT_ref, xT_ref, out_ref):
    r = randT_ref[...]
    mu = jnp.mean(r, axis=1, keepdims=True)
    var = jnp.sum((r - mu) ** 2, axis=1, keepdims=True) * (1.0 / (_D - 1))
    rn = r * lax.rsqrt(var)
    x = xT_ref[0]
    acc = lax.dot_general(rn, x, (((1,), (0,)), ((), ())),
                          preferred_element_type=jnp.float32)
    b = lax.bitcast_convert_type(acc, jnp.int32)
    mono = jnp.where(acc < 0, ~b, b | jnp.int32(-2147483648))
    col = lax.broadcasted_iota(jnp.int32, mono.shape, 1)
    mono24 = lax.shift_right_logical(mono, 8)
    out_ref[0] = jnp.where(col >= _L, jnp.int32(0xFFFFFF), mono24)


_sc_mesh = plsc.VectorSubcoreMesh(core_axis_name="c", subcore_axis_name="s")


@functools.partial(
    pl.kernel,
    mesh=_sc_mesh,
    compiler_params=pltpu.CompilerParams(
        needs_layout_passes=False, use_tc_tiling_on_sc=False),
    out_type=jax.ShapeDtypeStruct((_NTASK * 16,), jnp.float32),
    scratch_types=[
        pltpu.VMEM((_LP,), jnp.int32),
        pltpu.VMEM((_LP,), jnp.int32),
        pltpu.VMEM((_LP,), jnp.int32),
        pltpu.VMEM((_LP,), jnp.int32),
        pltpu.VMEM((_LP,), jnp.int32),
        pltpu.VMEM((_LP,), jnp.int32),
        pltpu.VMEM((_LP,), jnp.int32),
        pltpu.VMEM((_LP,), jnp.int32),
        pltpu.VMEM((_LP,), jnp.int32),
        pltpu.VMEM((_LP,), jnp.int32),
        pltpu.VMEM((4096,), jnp.int32),
        pltpu.VMEM((4096,), jnp.int32),
        pltpu.VMEM((_GCH, _DP), jnp.bfloat16),
        pltpu.VMEM((_GCH, _DP), jnp.bfloat16),
        pltpu.VMEM((_GCH, _DP), jnp.bfloat16),
        pltpu.VMEM((_TPT * 16,), jnp.float32),
        pltpu.SemaphoreType.DMA,
        pltpu.SemaphoreType.DMA,
        pltpu.SemaphoreType.DMA,
        pltpu.SemaphoreType.DMA,
        pltpu.SemaphoreType.DMA,
        pltpu.SemaphoreType.DMA,
    ],
)
def _sc_swd_kernel(keys_hbm, xp_hbm, yn_hbm, out_hbm,
                   kAx, kBx, pAx, pBx, kAy, kBy, pAy, pBy,
                   idxX, idxY, histx, histy,
                   b0, b1, b2, res,
                   sx0, sx1, sx2, sy0, sy1, sy2):
    wid = lax.axis_index("s") * 2 + lax.axis_index("c")
    lanes = lax.iota(jnp.int32, 16)
    ones = jnp.ones((16,), jnp.int32)
    zeros16 = jnp.zeros((16,), jnp.int32)
    gb0 = lanes * _CHUNK

    def radix_pass(xio, yio, shift, first, last, pbase):
        kinx, pinx, koutx, poutx = xio
        kiny, piny, kouty, pouty = yio

        def zb(i, c):
            histx[pl.ds(i * 16, 16)] = zeros16
            histy[pl.ds(i * 16, 16)] = zeros16
            return c
        lax.fori_loop(0, 256, zb, 0, unroll=4)

        def ph1(t, c):
            kx = plsc.load_gather(kinx, [gb0 + t])
            ky = plsc.load_gather(kiny, [gb0 + t])
            dgx = lax.shift_right_logical(kx, shift) & 255
            dgy = lax.shift_right_logical(ky, shift) & 255
            plsc.addupdate_scatter(histx, [dgx * 16 + lanes], ones)
            plsc.addupdate_scatter(histy, [dgy * 16 + lanes], ones)
            return c
        lax.fori_loop(0, _CHUNK, ph1, 0, unroll=2)

        def ph2(i, carry):
            cx, cy = carry
            vx = histx[pl.ds(i * 16, 16)]
            vy = histy[pl.ds(i * 16, 16)]
            incx = plsc.cumsum(vx)
            incy = plsc.cumsum(vy)
            histx[pl.ds(i * 16, 16)] = incx - vx + cx
            histy[pl.ds(i * 16, 16)] = incy - vy + cy
            return (cx + jnp.sum(vx), cy + jnp.sum(vy))
        lax.fori_loop(0, 256, ph2, (jnp.int32(0), jnp.int32(0)), unroll=2)

        def ph3(t, c):
            gidx = gb0 + t
            kx = plsc.load_gather(kinx, [gidx])
            ky = plsc.load_gather(kiny, [gidx])
            dgx = lax.shift_right_logical(kx, shift) & 255
            dgy = lax.shift_right_logical(ky, shift) & 255
            addrx = dgx * 16 + lanes
            addry = dgy * 16 + lanes
            offx = plsc.load_gather(histx, [addrx])
            offy = plsc.load_gather(histy, [addry])
            if first:
                px = gidx + pbase
                py = px
            else:
                px = plsc.load_gather(pinx, [gidx])
                py = plsc.load_gather(piny, [gidx])
            if not last:
                plsc.store_scatter(koutx, [offx], kx)
                plsc.store_scatter(kouty, [offy], ky)
            plsc.store_scatter(poutx, [offx], px)
            plsc.store_scatter(pouty, [offy], py)
            plsc.addupdate_scatter(histx, [addrx], ones)
            plsc.addupdate_scatter(histy, [addry], ones)
            return c
        lax.fori_loop(0, _CHUNK, ph3, 0, unroll=2)

    def sort_both(rowx, rowy, pbase):
        pltpu.sync_copy(keys_hbm.at[rowx], kAx)
        pltpu.sync_copy(keys_hbm.at[rowy], kAy)
        radix_pass((kAx, None, kBx, pBx), (kAy, None, kBy, pBy),
                   0, True, False, pbase)
        radix_pass((kBx, pBx, kAx, pAx), (kBy, pBy, kAy, pAy),
                   8, False, False, pbase)
        radix_pass((kAx, pAx, kBx, idxX), (kAy, pAy, kBy, idxY),
                   16, False, True, pbase)

    def task_body(r, c):
        task = wid * _TPT + r
        s = task // _NPROJ
        j = task - s * _NPROJ
        pbase = s * _LP
        sort_both((s * 2) * _NPROJ + j, (s * 2 + 1) * _NPROJ + j, pbase)

        bufs = ((b0, sx0, sy0), (b1, sx1, sy1), (b2, sx2, sy2))

        def issue_x(ci, slot):
            buf, sx, _ = bufs[slot]
            pltpu.async_copy(xp_hbm.at[idxX.at[pl.ds(ci * _GCH, _GCH)]],
                             buf, sx)

        def issue_yadd(ci, slot):
            buf, _, sy = bufs[slot]
            pltpu.async_copy(yn_hbm.at[idxY.at[pl.ds(ci * _GCH, _GCH)]],
                             buf, sy, add=True)

        def wait_x(slot):
            buf, sx, _ = bufs[slot]
            pltpu.make_async_copy(xp_hbm.at[pl.ds(0, _GCH)], buf, sx).wait()

        def wait_y(slot):
            buf, _, sy = bufs[slot]
            pltpu.make_async_copy(xp_hbm.at[pl.ds(0, _GCH)], buf, sy).wait()

        issue_x(0, 0)
        issue_x(1, 1)
        wait_x(0)
        issue_yadd(0, 0)

        def chunk_trip(i, acc):
            for b in (0, 1, 2):
                c = 3 * i + b
                m1 = (b + 1) % 3

                @pl.when(c + 1 < _NCH)
                def _():
                    wait_x(m1)
                    issue_yadd(c + 1, m1)

                buf = bufs[b][0]
                wait_y(b)

                def rowloop(rr, a):
                    d = buf[rr, pl.ds(0, 32)]
                    rs = jnp.abs(d)
                    for q in range(1, _DP // 32):
                        rs = rs + jnp.abs(buf[rr, pl.ds(q * 32, 32)])
                    lo, hi = plsc.unpack(
                        rs, format=plsc.PackFormat.INTERLEAVED)
                    return a + lo + hi
                acc = lax.fori_loop(0, _GCH, rowloop, acc)

                @pl.when(c + 2 < _NCH)
                def _():
                    issue_x(c + 2, (b + 2) % 3)
            return acc

        acc = lax.fori_loop(0, _NCH // 3, chunk_trip,
                            jnp.zeros((16,), jnp.float32))
        res[pl.ds(r * 16, 16)] = acc
        return c

    lax.fori_loop(0, _TPT, task_body, 0)
    pltpu.sync_copy(res, out_hbm.at[pl.ds(wid * _TPT * 16, _TPT * 16)])


def kernel(x, y, rand):
    xT = _patches_t(x)
    yT = _patches_t(y)
    xTp = jnp.pad(xT, ((0, 0), (0, 0), (0, _LP - _L)))
    yTp = jnp.pad(yT, ((0, 0), (0, 0), (0, _LP - _L)))
    xyT = jnp.stack([xTp, yTp], axis=1).reshape(2 * _B, _D, _LP)
    randT = jnp.transpose(rand)

    keys = pl.pallas_call(
        _proj_tc_kernel,
        grid=(2 * _B,),
        in_specs=[
            pl.BlockSpec((_NPROJ, _D), lambda i: (0, 0)),
            pl.BlockSpec((1, _D, _LP), lambda i: (i, 0, 0)),
        ],
        out_specs=pl.BlockSpec((1, _NPROJ, _LP), lambda i: (i, 0, 0)),
        out_shape=jax.ShapeDtypeStruct((2 * _B, _NPROJ, _LP), jnp.int32),
    )(randT, xyT)
    keys2 = keys.reshape(2 * _B * _NPROJ, _LP)

    xp = jnp.pad(jnp.transpose(xT, (0, 2, 1)),
                 ((0, 0), (0, _LP - _L), (0, _DP - _D))
                 ).reshape(_B * _LP, _DP).astype(jnp.bfloat16)
    yn = jnp.pad(jnp.transpose(-yT, (0, 2, 1)),
                 ((0, 0), (0, _LP - _L), (0, _DP - _D))
                 ).reshape(_B * _LP, _DP).astype(jnp.bfloat16)

    sums = _sc_swd_kernel(keys2, xp, yn)
    per_sample = sums.reshape(_B, _NPROJ * 16).sum(axis=1)
    return jnp.mean(per_sample / jnp.float32(_L * _D * _NPROJ))


# speedup vs baseline: 1.4572x; 1.0181x over previous
import functools

import jax
import jax.numpy as jnp
from jax import lax
from jax.experimental import pallas as pl
from jax.experimental.pallas import tpu as pltpu
from jax.experimental.pallas import tpu_sc as plsc

_PS = 7
_STRIDE = 2
_NPROJ = 256
_D = 147
_DP = 160
_L = 3721
_LP = 3840
_CHUNK = _LP // 16
_GCH = 128
_NCH = _LP // _GCH
_B = 4
_NTASK = _B * _NPROJ
_NTILE = 32
_TPT = _NTASK // _NTILE


def _patches_t(img):
    p = lax.conv_general_dilated_patches(
        img, filter_shape=(_PS, _PS), window_strides=(_STRIDE, _STRIDE),
        padding="VALID")
    return p.reshape(img.shape[0], _D, _L)


def _proj_tc_kernel(randT_ref, xT_ref, out_ref):
    r = randT_ref[...]
    mu = jnp.mean(r, axis=1, keepdims=True)
    var = jnp.sum((r - mu) ** 2, axis=1, keepdims=True) * (1.0 / (_D - 1))
    rn = r * lax.rsqrt(var)
    x = xT_ref[0]
    acc = lax.dot_general(rn, x, (((1,), (0,)), ((), ())),
                          preferred_element_type=jnp.float32)
    b = lax.bitcast_convert_type(acc, jnp.int32)
    mono = jnp.where(acc < 0, ~b, b | jnp.int32(-2147483648))
    col = lax.broadcasted_iota(jnp.int32, mono.shape, 1)
    mono24 = lax.shift_right_logical(mono, 8)
    out_ref[0] = jnp.where(col >= _L, jnp.int32(0xFFFFFF), mono24)


_sc_mesh = plsc.VectorSubcoreMesh(core_axis_name="c", subcore_axis_name="s")


@functools.partial(
    pl.kernel,
    mesh=_sc_mesh,
    compiler_params=pltpu.CompilerParams(
        needs_layout_passes=False, use_tc_tiling_on_sc=False),
    out_type=jax.ShapeDtypeStruct((_NTASK * 16,), jnp.float32),
    scratch_types=[
        pltpu.VMEM((_LP,), jnp.int32),
        pltpu.VMEM((_LP,), jnp.int32),
        pltpu.VMEM((_LP,), jnp.int32),
        pltpu.VMEM((_LP,), jnp.int32),
        pltpu.VMEM((_LP,), jnp.int32),
        pltpu.VMEM((_LP,), jnp.int32),
        pltpu.VMEM((_LP,), jnp.int32),
        pltpu.VMEM((_LP,), jnp.int32),
        pltpu.VMEM((_LP,), jnp.int32),
        pltpu.VMEM((_LP,), jnp.int32),
        pltpu.VMEM((4096,), jnp.int32),
        pltpu.VMEM((4096,), jnp.int32),
        pltpu.VMEM((_GCH, _DP), jnp.bfloat16),
        pltpu.VMEM((_GCH, _DP), jnp.bfloat16),
        pltpu.VMEM((_GCH, _DP), jnp.bfloat16),
        pltpu.VMEM((_TPT * 16,), jnp.float32),
        pltpu.SemaphoreType.DMA,
        pltpu.SemaphoreType.DMA,
        pltpu.SemaphoreType.DMA,
        pltpu.SemaphoreType.DMA,
        pltpu.SemaphoreType.DMA,
        pltpu.SemaphoreType.DMA,
    ],
)
def _sc_swd_kernel(keys_hbm, xp_hbm, yn_hbm, out_hbm,
                   kAx, kBx, pAx, pBx, kAy, kBy, pAy, pBy,
                   idxX, idxY, histx, histy,
                   b0, b1, b2, res,
                   sx0, sx1, sx2, sy0, sy1, sy2):
    wid = lax.axis_index("s") * 2 + lax.axis_index("c")
    lanes = lax.iota(jnp.int32, 16)
    ones = jnp.ones((16,), jnp.int32)
    zeros16 = jnp.zeros((16,), jnp.int32)
    gb0 = lanes * _CHUNK

    def radix_pass(xio, yio, shift, first, last, pbase):
        kinx, pinx, koutx, poutx = xio
        kiny, piny, kouty, pouty = yio

        def zb(i, c):
            histx[pl.ds(i * 16, 16)] = zeros16
            histy[pl.ds(i * 16, 16)] = zeros16
            return c
        lax.fori_loop(0, 256, zb, 0, unroll=4)

        def ph1(t, c):
            kx = plsc.load_gather(kinx, [gb0 + t])
            ky = plsc.load_gather(kiny, [gb0 + t])
            dgx = lax.shift_right_logical(kx, shift) & 255
            dgy = lax.shift_right_logical(ky, shift) & 255
            plsc.addupdate_scatter(histx, [dgx * 16 + lanes], ones)
            plsc.addupdate_scatter(histy, [dgy * 16 + lanes], ones)
            return c
        lax.fori_loop(0, _CHUNK, ph1, 0, unroll=4)

        def ph2(i, carry):
            cx, cy = carry
            vx = histx[pl.ds(i * 16, 16)]
            vy = histy[pl.ds(i * 16, 16)]
            incx = plsc.cumsum(vx)
            incy = plsc.cumsum(vy)
            histx[pl.ds(i * 16, 16)] = incx - vx + cx
            histy[pl.ds(i * 16, 16)] = incy - vy + cy
            return (cx + jnp.sum(vx), cy + jnp.sum(vy))
        lax.fori_loop(0, 256, ph2, (jnp.int32(0), jnp.int32(0)), unroll=4)

        def ph3(t, c):
            gidx = gb0 + t
            kx = plsc.load_gather(kinx, [gidx])
            ky = plsc.load_gather(kiny, [gidx])
            dgx = lax.shift_right_logical(kx, shift) & 255
            dgy = lax.shift_right_logical(ky, shift) & 255
            addrx = dgx * 16 + lanes
            addry = dgy * 16 + lanes
            offx = plsc.load_gather(histx, [addrx])
            offy = plsc.load_gather(histy, [addry])
            if first:
                px = gidx + pbase
                py = px
            else:
                px = plsc.load_gather(pinx, [gidx])
                py = plsc.load_gather(piny, [gidx])
            if not last:
                plsc.store_scatter(koutx, [offx], kx)
                plsc.store_scatter(kouty, [offy], ky)
            plsc.store_scatter(poutx, [offx], px)
            plsc.store_scatter(pouty, [offy], py)
            plsc.addupdate_scatter(histx, [addrx], ones)
            plsc.addupdate_scatter(histy, [addry], ones)
            return c
        lax.fori_loop(0, _CHUNK, ph3, 0, unroll=4)

    def sort_both(rowx, rowy, pbase):
        pltpu.sync_copy(keys_hbm.at[rowx], kAx)
        pltpu.sync_copy(keys_hbm.at[rowy], kAy)
        radix_pass((kAx, None, kBx, pBx), (kAy, None, kBy, pBy),
                   0, True, False, pbase)
        radix_pass((kBx, pBx, kAx, pAx), (kBy, pBy, kAy, pAy),
                   8, False, False, pbase)
        radix_pass((kAx, pAx, kBx, idxX), (kAy, pAy, kBy, idxY),
                   16, False, True, pbase)

    def task_body(r, c):
        task = wid * _TPT + r
        s = task // _NPROJ
        j = task - s * _NPROJ
        pbase = s * _LP
        sort_both((s * 2) * _NPROJ + j, (s * 2 + 1) * _NPROJ + j, pbase)

        bufs = ((b0, sx0, sy0), (b1, sx1, sy1), (b2, sx2, sy2))

        def issue_x(ci, slot):
            buf, sx, _ = bufs[slot]
            pltpu.async_copy(xp_hbm.at[idxX.at[pl.ds(ci * _GCH, _GCH)]],
                             buf, sx)

        def issue_yadd(ci, slot):
            buf, _, sy = bufs[slot]
            pltpu.async_copy(yn_hbm.at[idxY.at[pl.ds(ci * _GCH, _GCH)]],
                             buf, sy, add=True)

        def wait_x(slot):
            buf, sx, _ = bufs[slot]
            pltpu.make_async_copy(xp_hbm.at[pl.ds(0, _GCH)], buf, sx).wait()

        def wait_y(slot):
            buf, _, sy = bufs[slot]
            pltpu.make_async_copy(xp_hbm.at[pl.ds(0, _GCH)], buf, sy).wait()

        issue_x(0, 0)
        issue_x(1, 1)
        wait_x(0)
        issue_yadd(0, 0)

        def chunk_trip(i, acc):
            for b in (0, 1, 2):
                c = 3 * i + b
                m1 = (b + 1) % 3

                @pl.when(c + 1 < _NCH)
                def _():
                    wait_x(m1)
                    issue_yadd(c + 1, m1)

                buf = bufs[b][0]
                wait_y(b)

                def rowloop(rr, a):
                    d = buf[rr, pl.ds(0, 32)]
                    rs = jnp.abs(d)
                    for q in range(1, _DP // 32):
                        rs = rs + jnp.abs(buf[rr, pl.ds(q * 32, 32)])
                    lo, hi = plsc.unpack(
                        rs, format=plsc.PackFormat.INTERLEAVED)
                    return a + lo + hi
                acc = lax.fori_loop(0, _GCH, rowloop, acc, unroll=2)

                @pl.when(c + 2 < _NCH)
                def _():
                    issue_x(c + 2, (b + 2) % 3)
            return acc

        acc = lax.fori_loop(0, _NCH // 3, chunk_trip,
                            jnp.zeros((16,), jnp.float32))
        res[pl.ds(r * 16, 16)] = acc
        return c

    lax.fori_loop(0, _TPT, task_body, 0)
    pltpu.sync_copy(res, out_hbm.at[pl.ds(wid * _TPT * 16, _TPT * 16)])


def kernel(x, y, rand):
    xT = _patches_t(x)
    yT = _patches_t(y)
    xTp = jnp.pad(xT, ((0, 0), (0, 0), (0, _LP - _L)))
    yTp = jnp.pad(yT, ((0, 0), (0, 0), (0, _LP - _L)))
    xyT = jnp.stack([xTp, yTp], axis=1).reshape(2 * _B, _D, _LP)
    randT = jnp.transpose(rand)

    keys = pl.pallas_call(
        _proj_tc_kernel,
        grid=(2 * _B,),
        in_specs=[
            pl.BlockSpec((_NPROJ, _D), lambda i: (0, 0)),
            pl.BlockSpec((1, _D, _LP), lambda i: (i, 0, 0)),
        ],
        out_specs=pl.BlockSpec((1, _NPROJ, _LP), lambda i: (i, 0, 0)),
        out_shape=jax.ShapeDtypeStruct((2 * _B, _NPROJ, _LP), jnp.int32),
    )(randT, xyT)
    keys2 = keys.reshape(2 * _B * _NPROJ, _LP)

    xp = jnp.pad(jnp.transpose(xT, (0, 2, 1)),
                 ((0, 0), (0, _LP - _L), (0, _DP - _D))
                 ).reshape(_B * _LP, _DP).astype(jnp.bfloat16)
    yn = jnp.pad(jnp.transpose(-yT, (0, 2, 1)),
                 ((0, 0), (0, _LP - _L), (0, _DP - _D))
                 ).reshape(_B * _LP, _DP).astype(jnp.bfloat16)

    sums = _sc_swd_kernel(keys2, xp, yn)
    per_sample = sums.reshape(_B, _NPROJ * 16).sum(axis=1)
    return jnp.mean(per_sample / jnp.float32(_L * _D * _NPROJ))


# speedup vs baseline: 1.4709x; 1.0094x over previous
import functools

import jax
import jax.numpy as jnp
from jax import lax
from jax.experimental import pallas as pl
from jax.experimental.pallas import tpu as pltpu
from jax.experimental.pallas import tpu_sc as plsc

_PS = 7
_STRIDE = 2
_NPROJ = 256
_D = 147
_DP = 160
_L = 3721
_LP = 4096
_CHUNK = _LP // 16
_GCH = 128
_NCH = _LP // _GCH
_B = 4
_NTASK = _B * _NPROJ
_NTILE = 32
_TPT = _NTASK // _NTILE


def _patches_t(img):
    p = lax.conv_general_dilated_patches(
        img, filter_shape=(_PS, _PS), window_strides=(_STRIDE, _STRIDE),
        padding="VALID")
    return p.reshape(img.shape[0], _D, _L)


def _proj_tc_kernel(randT_ref, xT_ref, out_ref):
    r = randT_ref[...]
    mu = jnp.mean(r, axis=1, keepdims=True)
    var = jnp.sum((r - mu) ** 2, axis=1, keepdims=True) * (1.0 / (_D - 1))
    rn = r * lax.rsqrt(var)
    x = xT_ref[0]
    acc = lax.dot_general(rn, x, (((1,), (0,)), ((), ())),
                          preferred_element_type=jnp.float32)
    b = lax.bitcast_convert_type(acc, jnp.int32)
    mono = jnp.where(acc < 0, ~b, b | jnp.int32(-2147483648))
    col = lax.broadcasted_iota(jnp.int32, mono.shape, 1)
    mono24 = lax.shift_right_logical(mono, 8)
    out_ref[0] = jnp.where(col >= _L, jnp.int32(0xFFFFFF), mono24)


_sc_mesh = plsc.VectorSubcoreMesh(core_axis_name="c", subcore_axis_name="s")


@functools.partial(
    pl.kernel,
    mesh=_sc_mesh,
    compiler_params=pltpu.CompilerParams(
        needs_layout_passes=False, use_tc_tiling_on_sc=False),
    out_type=jax.ShapeDtypeStruct((_NTASK * 16,), jnp.float32),
    scratch_types=[
        pltpu.VMEM((_LP,), jnp.int32),
        pltpu.VMEM((_LP,), jnp.int32),
        pltpu.VMEM((_LP,), jnp.int32),
        pltpu.VMEM((_LP,), jnp.int32),
        pltpu.VMEM((_LP,), jnp.int32),
        pltpu.VMEM((_LP,), jnp.int32),
        pltpu.VMEM((_LP,), jnp.int32),
        pltpu.VMEM((_LP,), jnp.int32),
        pltpu.VMEM((_LP,), jnp.int32),
        pltpu.VMEM((_LP,), jnp.int32),
        pltpu.VMEM((4096,), jnp.int32),
        pltpu.VMEM((4096,), jnp.int32),
        pltpu.VMEM((_GCH, _DP), jnp.bfloat16),
        pltpu.VMEM((_GCH, _DP), jnp.bfloat16),
        pltpu.VMEM((_GCH, _DP), jnp.bfloat16),
        pltpu.VMEM((_GCH, _DP), jnp.bfloat16),
        pltpu.VMEM((_TPT * 16,), jnp.float32),
        pltpu.SemaphoreType.DMA,
        pltpu.SemaphoreType.DMA,
        pltpu.SemaphoreType.DMA,
        pltpu.SemaphoreType.DMA,
        pltpu.SemaphoreType.DMA,
        pltpu.SemaphoreType.DMA,
        pltpu.SemaphoreType.DMA,
        pltpu.SemaphoreType.DMA,
    ],
)
def _sc_swd_kernel(keys_hbm, xp_hbm, yn_hbm, out_hbm,
                   kAx, kBx, pAx, pBx, kAy, kBy, pAy, pBy,
                   idxX, idxY, histx, histy,
                   b0, b1, b2, b3, res,
                   sx0, sx1, sx2, sx3, sy0, sy1, sy2, sy3):
    wid = lax.axis_index("s") * 2 + lax.axis_index("c")
    lanes = lax.iota(jnp.int32, 16)
    ones = jnp.ones((16,), jnp.int32)
    zeros16 = jnp.zeros((16,), jnp.int32)
    gb0 = lanes * _CHUNK

    def radix_pass(xio, yio, shift, first, last, pbase):
        kinx, pinx, koutx, poutx = xio
        kiny, piny, kouty, pouty = yio

        def zb(i, c):
            histx[pl.ds(i * 16, 16)] = zeros16
            histy[pl.ds(i * 16, 16)] = zeros16
            return c
        lax.fori_loop(0, 256, zb, 0, unroll=4)

        def ph1(t, c):
            if first:
                kx = plsc.load_gather(kinx, [gb0 + t])
                ky = plsc.load_gather(kiny, [gb0 + t])
            else:
                kx = kinx[pl.ds(t * 16, 16)]
                ky = kiny[pl.ds(t * 16, 16)]
            dgx = lax.shift_right_logical(kx, shift) & 255
            dgy = lax.shift_right_logical(ky, shift) & 255
            plsc.addupdate_scatter(histx, [dgx * 16 + lanes], ones)
            plsc.addupdate_scatter(histy, [dgy * 16 + lanes], ones)
            return c
        lax.fori_loop(0, _CHUNK, ph1, 0, unroll=4)

        def ph2(i, carry):
            cx, cy = carry
            vx = histx[pl.ds(i * 16, 16)]
            vy = histy[pl.ds(i * 16, 16)]
            incx = plsc.cumsum(vx)
            incy = plsc.cumsum(vy)
            histx[pl.ds(i * 16, 16)] = incx - vx + cx
            histy[pl.ds(i * 16, 16)] = incy - vy + cy
            return (cx + jnp.sum(vx), cy + jnp.sum(vy))
        lax.fori_loop(0, 256, ph2, (jnp.int32(0), jnp.int32(0)), unroll=4)

        def swz(off):
            return lax.shift_left(off & 255, 4) | lax.shift_right_logical(
                off, 8)

        def ph3(t, c):
            if first:
                gidx = gb0 + t
                kx = plsc.load_gather(kinx, [gidx])
                ky = plsc.load_gather(kiny, [gidx])
                px = gidx + pbase
                py = px
            else:
                kx = kinx[pl.ds(t * 16, 16)]
                ky = kiny[pl.ds(t * 16, 16)]
                px = pinx[pl.ds(t * 16, 16)]
                py = piny[pl.ds(t * 16, 16)]
            dgx = lax.shift_right_logical(kx, shift) & 255
            dgy = lax.shift_right_logical(ky, shift) & 255
            addrx = dgx * 16 + lanes
            addry = dgy * 16 + lanes
            offx = plsc.load_gather(histx, [addrx])
            offy = plsc.load_gather(histy, [addry])
            if last:
                plsc.store_scatter(poutx, [offx], px)
                plsc.store_scatter(pouty, [offy], py)
            else:
                sx_ = swz(offx)
                sy_ = swz(offy)
                plsc.store_scatter(koutx, [sx_], kx)
                plsc.store_scatter(kouty, [sy_], ky)
                plsc.store_scatter(poutx, [sx_], px)
                plsc.store_scatter(pouty, [sy_], py)
            plsc.addupdate_scatter(histx, [addrx], ones)
            plsc.addupdate_scatter(histy, [addry], ones)
            return c
        lax.fori_loop(0, _CHUNK, ph3, 0, unroll=4)

    def sort_both(rowx, rowy, pbase):
        pltpu.sync_copy(keys_hbm.at[rowx], kAx)
        pltpu.sync_copy(keys_hbm.at[rowy], kAy)
        radix_pass((kAx, None, kBx, pBx), (kAy, None, kBy, pBy),
                   0, True, False, pbase)
        radix_pass((kBx, pBx, kAx, pAx), (kBy, pBy, kAy, pAy),
                   8, False, False, pbase)
        radix_pass((kAx, pAx, kBx, idxX), (kAy, pAy, kBy, idxY),
                   16, False, True, pbase)

    def task_body(r, c):
        task = wid * _TPT + r
        s = task // _NPROJ
        j = task - s * _NPROJ
        pbase = s * _LP
        sort_both((s * 2) * _NPROJ + j, (s * 2 + 1) * _NPROJ + j, pbase)

        bufs = ((b0, sx0, sy0), (b1, sx1, sy1), (b2, sx2, sy2),
                (b3, sx3, sy3))

        def issue_x(ci, slot):
            buf, sx, _ = bufs[slot]
            pltpu.async_copy(xp_hbm.at[idxX.at[pl.ds(ci * _GCH, _GCH)]],
                             buf, sx)

        def issue_yadd(ci, slot):
            buf, _, sy = bufs[slot]
            pltpu.async_copy(yn_hbm.at[idxY.at[pl.ds(ci * _GCH, _GCH)]],
                             buf, sy, add=True)

        def wait_x(slot):
            buf, sx, _ = bufs[slot]
            pltpu.make_async_copy(xp_hbm.at[pl.ds(0, _GCH)], buf, sx).wait()

        def wait_y(slot):
            buf, _, sy = bufs[slot]
            pltpu.make_async_copy(xp_hbm.at[pl.ds(0, _GCH)], buf, sy).wait()

        issue_x(0, 0)
        issue_x(1, 1)
        wait_x(0)
        issue_yadd(0, 0)

        def chunk_quad(i, accs):
            for b in (0, 1, 2, 3):
                c = 4 * i + b
                m1 = (b + 1) % 4

                @pl.when(c + 1 < _NCH)
                def _():
                    wait_x(m1)
                    issue_yadd(c + 1, m1)

                buf = bufs[b][0]
                wait_y(b)

                def rowloop(rr, carry):
                    a0, a1, a2, a3 = carry
                    d0 = jnp.abs(buf[rr, pl.ds(0, 32)])
                    d1 = jnp.abs(buf[rr, pl.ds(32, 32)])
                    d2 = jnp.abs(buf[rr, pl.ds(64, 32)])
                    d3 = jnp.abs(buf[rr, pl.ds(96, 32)])
                    d4 = jnp.abs(buf[rr, pl.ds(128, 32)])
                    rs = (d0 + d1) + (d2 + d3) + d4
                    lo, hi = plsc.unpack(
                        rs, format=plsc.PackFormat.INTERLEAVED)
                    return (a1, a2, a3, a0 + lo + hi)
                accs = lax.fori_loop(0, _GCH, rowloop, accs, unroll=4)

                @pl.when(c + 2 < _NCH)
                def _():
                    issue_x(c + 2, (b + 2) % 4)
            return accs

        zf = jnp.zeros((16,), jnp.float32)
        accs = lax.fori_loop(0, _NCH // 4, chunk_quad, (zf, zf, zf, zf))
        acc = (accs[0] + accs[1]) + (accs[2] + accs[3])
        res[pl.ds(r * 16, 16)] = acc
        return c

    lax.fori_loop(0, _TPT, task_body, 0)
    pltpu.sync_copy(res, out_hbm.at[pl.ds(wid * _TPT * 16, _TPT * 16)])


def kernel(x, y, rand):
    xT = _patches_t(x)
    yT = _patches_t(y)
    xTp = jnp.pad(xT, ((0, 0), (0, 0), (0, _LP - _L)))
    yTp = jnp.pad(yT, ((0, 0), (0, 0), (0, _LP - _L)))
    xyT = jnp.stack([xTp, yTp], axis=1).reshape(2 * _B, _D, _LP)
    randT = jnp.transpose(rand)

    keys = pl.pallas_call(
        _proj_tc_kernel,
        grid=(2 * _B,),
        in_specs=[
            pl.BlockSpec((_NPROJ, _D), lambda i: (0, 0)),
            pl.BlockSpec((1, _D, _LP), lambda i: (i, 0, 0)),
        ],
        out_specs=pl.BlockSpec((1, _NPROJ, _LP), lambda i: (i, 0, 0)),
        out_shape=jax.ShapeDtypeStruct((2 * _B, _NPROJ, _LP), jnp.int32),
    )(randT, xyT)
    keys2 = keys.reshape(2 * _B * _NPROJ, _LP)

    xp = jnp.pad(jnp.transpose(xT, (0, 2, 1)),
                 ((0, 0), (0, _LP - _L), (0, _DP - _D))
                 ).reshape(_B * _LP, _DP).astype(jnp.bfloat16)
    yn = jnp.pad(jnp.transpose(-yT, (0, 2, 1)),
                 ((0, 0), (0, _LP - _L), (0, _DP - _D))
                 ).reshape(_B * _LP, _DP).astype(jnp.bfloat16)

    sums = _sc_swd_kernel(keys2, xp, yn)
    per_sample = sums.reshape(_B, _NPROJ * 16).sum(axis=1)
    return jnp.mean(per_sample / jnp.float32(_L * _D * _NPROJ))


# speedup vs baseline: 1.5955x; 1.0847x over previous
import functools

import jax
import jax.numpy as jnp
from jax import lax
from jax.experimental import pallas as pl
from jax.experimental.pallas import tpu as pltpu
from jax.experimental.pallas import tpu_sc as plsc

_PS = 7
_STRIDE = 2
_NPROJ = 256
_D = 147
_DP = 160
_L = 3721
_LP = 4096
_CHUNK = _LP // 16
_GCH = 128
_NCH = _LP // _GCH
_B = 4
_NTASK = _B * _NPROJ
_NTILE = 32
_TPT = _NTASK // _NTILE


def _patches_t(img):
    p = lax.conv_general_dilated_patches(
        img, filter_shape=(_PS, _PS), window_strides=(_STRIDE, _STRIDE),
        padding="VALID")
    return p.reshape(img.shape[0], _D, _L)


def _proj_tc_kernel(randT_ref, xT_ref, out_ref):
    r = randT_ref[...]
    mu = jnp.mean(r, axis=1, keepdims=True)
    var = jnp.sum((r - mu) ** 2, axis=1, keepdims=True) * (1.0 / (_D - 1))
    rn = r * lax.rsqrt(var)
    x = xT_ref[0]
    acc = lax.dot_general(rn, x, (((1,), (0,)), ((), ())),
                          preferred_element_type=jnp.float32)
    b = lax.bitcast_convert_type(acc, jnp.int32)
    mono = jnp.where(acc < 0, ~b, b | jnp.int32(-2147483648))
    col = lax.broadcasted_iota(jnp.int32, mono.shape, 1)
    mono24 = lax.shift_right_logical(mono, 8)
    out_ref[0] = jnp.where(col >= _L, jnp.int32(0xFFFFFF), mono24)


_sc_mesh = plsc.VectorSubcoreMesh(core_axis_name="c", subcore_axis_name="s")


@functools.partial(
    pl.kernel,
    mesh=_sc_mesh,
    compiler_params=pltpu.CompilerParams(
        needs_layout_passes=False, use_tc_tiling_on_sc=False),
    out_type=jax.ShapeDtypeStruct((_NTASK * 16,), jnp.float32),
    scratch_types=[
        pltpu.VMEM((_LP,), jnp.int32),
        pltpu.VMEM((_LP,), jnp.int32),
        pltpu.VMEM((_LP,), jnp.int32),
        pltpu.VMEM((_LP,), jnp.int32),
        pltpu.VMEM((_LP,), jnp.int32),
        pltpu.VMEM((_LP,), jnp.int32),
        pltpu.VMEM((_LP,), jnp.int32),
        pltpu.VMEM((_LP,), jnp.int32),
        pltpu.VMEM((_LP,), jnp.int32),
        pltpu.VMEM((_LP,), jnp.int32),
        pltpu.VMEM((4096,), jnp.int32),
        pltpu.VMEM((4096,), jnp.int32),
        pltpu.SMEM((256,), jnp.int32),
        pltpu.SMEM((256,), jnp.int32),
        pltpu.VMEM((_GCH, _DP), jnp.bfloat16),
        pltpu.VMEM((_GCH, _DP), jnp.bfloat16),
        pltpu.VMEM((_GCH, _DP), jnp.bfloat16),
        pltpu.VMEM((_GCH, _DP), jnp.bfloat16),
        pltpu.VMEM((_TPT * 16,), jnp.float32),
        pltpu.SemaphoreType.DMA,
        pltpu.SemaphoreType.DMA,
        pltpu.SemaphoreType.DMA,
        pltpu.SemaphoreType.DMA,
        pltpu.SemaphoreType.DMA,
        pltpu.SemaphoreType.DMA,
        pltpu.SemaphoreType.DMA,
        pltpu.SemaphoreType.DMA,
    ],
)
def _sc_swd_kernel(keys_hbm, xp_hbm, yn_hbm, out_hbm,
                   kAx, kBx, pAx, pBx, kAy, kBy, pAy, pBy,
                   idxX, idxY, histx, histy, smx, smy,
                   b0, b1, b2, b3, res,
                   sx0, sx1, sx2, sx3, sy0, sy1, sy2, sy3):
    wid = lax.axis_index("s") * 2 + lax.axis_index("c")
    lanes = lax.iota(jnp.int32, 16)
    ones = jnp.ones((16,), jnp.int32)
    zeros16 = jnp.zeros((16,), jnp.int32)
    gb0 = lanes * _CHUNK

    def radix_pass(xio, yio, shift, first, last, pbase):
        kinx, pinx, koutx, poutx = xio
        kiny, piny, kouty, pouty = yio

        def zb(i, c):
            histx[pl.ds(i * 16, 16)] = zeros16
            histy[pl.ds(i * 16, 16)] = zeros16
            return c
        lax.fori_loop(0, 256, zb, 0, unroll=4)

        _G = 4

        def load_keys(t):
            if first:
                return (plsc.load_gather(kinx, [gb0 + t]),
                        plsc.load_gather(kiny, [gb0 + t]))
            return (kinx[pl.ds(t * 16, 16)], kiny[pl.ds(t * 16, 16)])

        def ph1(i, c):
            t0 = i * _G
            ks = [load_keys(t0 + u) for u in range(_G)]
            for kx, ky in ks:
                dgx = lax.shift_right_logical(kx, shift) & 255
                dgy = lax.shift_right_logical(ky, shift) & 255
                plsc.addupdate_scatter(histx, [dgx * 16 + lanes], ones)
                plsc.addupdate_scatter(histy, [dgy * 16 + lanes], ones)
            return c
        lax.fori_loop(0, _CHUNK // _G, ph1, 0)

        def p2a(i, c):
            smx[i] = jnp.sum(histx[pl.ds(i * 16, 16)])
            smy[i] = jnp.sum(histy[pl.ds(i * 16, 16)])
            return c
        lax.fori_loop(0, 256, p2a, 0, unroll=2)

        def p2b(i, carry):
            cx, cy = carry
            tx = smx[i]
            ty = smy[i]
            smx[i] = cx
            smy[i] = cy
            return (cx + tx, cy + ty)
        lax.fori_loop(0, 256, p2b, (jnp.int32(0), jnp.int32(0)))

        def p2c(i, c):
            vx = histx[pl.ds(i * 16, 16)]
            vy = histy[pl.ds(i * 16, 16)]
            histx[pl.ds(i * 16, 16)] = plsc.cumsum(vx) - vx + smx[i]
            histy[pl.ds(i * 16, 16)] = plsc.cumsum(vy) - vy + smy[i]
            return c
        lax.fori_loop(0, 256, p2c, 0, unroll=2)

        def swz(off):
            return lax.shift_left(off & 255, 4) | lax.shift_right_logical(
                off, 8)

        def ph3(i, c):
            t0 = i * _G
            ks = [load_keys(t0 + u) for u in range(_G)]
            if first:
                ps = [(gb0 + t0 + u + pbase,) * 2 for u in range(_G)]
            else:
                ps = [(pinx[pl.ds((t0 + u) * 16, 16)],
                       piny[pl.ds((t0 + u) * 16, 16)]) for u in range(_G)]
            ax = [None] * _G
            ay = [None] * _G
            for u, (kx, ky) in enumerate(ks):
                ax[u] = ((lax.shift_right_logical(kx, shift) & 255) * 16
                         + lanes)
                ay[u] = ((lax.shift_right_logical(ky, shift) & 255) * 16
                         + lanes)
            ox = [plsc.load_gather(histx, [a]) for a in ax]
            oy = [plsc.load_gather(histy, [a]) for a in ay]
            for u in range(1, _G):
                cx = (ax[u] == ax[0]).astype(jnp.int32)
                cy = (ay[u] == ay[0]).astype(jnp.int32)
                for v in range(1, u):
                    cx = cx + (ax[u] == ax[v]).astype(jnp.int32)
                    cy = cy + (ay[u] == ay[v]).astype(jnp.int32)
                ox[u] = ox[u] + cx
                oy[u] = oy[u] + cy
            for u, (kx, ky) in enumerate(ks):
                px, py = ps[u]
                if last:
                    plsc.store_scatter(poutx, [ox[u]], px)
                    plsc.store_scatter(pouty, [oy[u]], py)
                else:
                    sx_ = swz(ox[u])
                    sy_ = swz(oy[u])
                    plsc.store_scatter(koutx, [sx_], kx)
                    plsc.store_scatter(kouty, [sy_], ky)
                    plsc.store_scatter(poutx, [sx_], px)
                    plsc.store_scatter(pouty, [sy_], py)
            for u in range(_G):
                plsc.addupdate_scatter(histx, [ax[u]], ones)
                plsc.addupdate_scatter(histy, [ay[u]], ones)
            return c
        lax.fori_loop(0, _CHUNK // _G, ph3, 0)

    def sort_both(rowx, rowy, pbase):
        pltpu.sync_copy(keys_hbm.at[rowx], kAx)
        pltpu.sync_copy(keys_hbm.at[rowy], kAy)
        radix_pass((kAx, None, kBx, pBx), (kAy, None, kBy, pBy),
                   0, True, False, pbase)
        radix_pass((kBx, pBx, kAx, pAx), (kBy, pBy, kAy, pAy),
                   8, False, False, pbase)
        radix_pass((kAx, pAx, kBx, idxX), (kAy, pAy, kBy, idxY),
                   16, False, True, pbase)

    def task_body(r, c):
        task = wid * _TPT + r
        s = task // _NPROJ
        j = task - s * _NPROJ
        pbase = s * _LP
        sort_both((s * 2) * _NPROJ + j, (s * 2 + 1) * _NPROJ + j, pbase)

        bufs = ((b0, sx0, sy0), (b1, sx1, sy1), (b2, sx2, sy2),
                (b3, sx3, sy3))

        def issue_x(ci, slot):
            buf, sx, _ = bufs[slot]
            pltpu.async_copy(xp_hbm.at[idxX.at[pl.ds(ci * _GCH, _GCH)]],
                             buf, sx)

        def issue_yadd(ci, slot):
            buf, _, sy = bufs[slot]
            pltpu.async_copy(yn_hbm.at[idxY.at[pl.ds(ci * _GCH, _GCH)]],
                             buf, sy, add=True)

        def wait_x(slot):
            buf, sx, _ = bufs[slot]
            pltpu.make_async_copy(xp_hbm.at[pl.ds(0, _GCH)], buf, sx).wait()

        def wait_y(slot):
            buf, _, sy = bufs[slot]
            pltpu.make_async_copy(xp_hbm.at[pl.ds(0, _GCH)], buf, sy).wait()

        issue_x(0, 0)
        issue_x(1, 1)
        wait_x(0)
        issue_yadd(0, 0)

        def chunk_quad(i, accs):
            for b in (0, 1, 2, 3):
                c = 4 * i + b
                m1 = (b + 1) % 4

                @pl.when(c + 1 < _NCH)
                def _():
                    wait_x(m1)
                    issue_yadd(c + 1, m1)

                buf = bufs[b][0]
                wait_y(b)

                def rowloop(ri, carry):
                    a0, a1, a2, a3 = carry
                    r0 = ri * 2
                    ds = [buf[r0 + (q // 5), pl.ds((q % 5) * 32, 32)]
                          for q in range(10)]
                    e = [jnp.abs(d) for d in ds]
                    rs0 = ((e[0] + e[1]) + (e[2] + e[3])) + e[4]
                    rs1 = ((e[5] + e[6]) + (e[7] + e[8])) + e[9]
                    lo0, hi0 = plsc.unpack(
                        rs0, format=plsc.PackFormat.INTERLEAVED)
                    lo1, hi1 = plsc.unpack(
                        rs1, format=plsc.PackFormat.INTERLEAVED)
                    return (a0 + lo0, a1 + hi0, a2 + lo1, a3 + hi1)
                accs = lax.fori_loop(0, _GCH // 2, rowloop, accs, unroll=2)

                @pl.when(c + 2 < _NCH)
                def _():
                    issue_x(c + 2, (b + 2) % 4)
            return accs

        zf = jnp.zeros((16,), jnp.float32)
        accs = lax.fori_loop(0, _NCH // 4, chunk_quad, (zf, zf, zf, zf))
        acc = (accs[0] + accs[1]) + (accs[2] + accs[3])
        res[pl.ds(r * 16, 16)] = acc
        return c

    lax.fori_loop(0, _TPT, task_body, 0)
    pltpu.sync_copy(res, out_hbm.at[pl.ds(wid * _TPT * 16, _TPT * 16)])


def kernel(x, y, rand):
    xT = _patches_t(x)
    yT = _patches_t(y)
    xTp = jnp.pad(xT, ((0, 0), (0, 0), (0, _LP - _L)))
    yTp = jnp.pad(yT, ((0, 0), (0, 0), (0, _LP - _L)))
    xyT = jnp.stack([xTp, yTp], axis=1).reshape(2 * _B, _D, _LP)
    randT = jnp.transpose(rand)

    keys = pl.pallas_call(
        _proj_tc_kernel,
        grid=(2 * _B,),
        in_specs=[
            pl.BlockSpec((_NPROJ, _D), lambda i: (0, 0)),
            pl.BlockSpec((1, _D, _LP), lambda i: (i, 0, 0)),
        ],
        out_specs=pl.BlockSpec((1, _NPROJ, _LP), lambda i: (i, 0, 0)),
        out_shape=jax.ShapeDtypeStruct((2 * _B, _NPROJ, _LP), jnp.int32),
    )(randT, xyT)
    keys2 = keys.reshape(2 * _B * _NPROJ, _LP)

    xp = jnp.pad(jnp.transpose(xT, (0, 2, 1)),
                 ((0, 0), (0, _LP - _L), (0, _DP - _D))
                 ).reshape(_B * _LP, _DP).astype(jnp.bfloat16)
    yn = jnp.pad(jnp.transpose(-yT, (0, 2, 1)),
                 ((0, 0), (0, _LP - _L), (0, _DP - _D))
                 ).reshape(_B * _LP, _DP).astype(jnp.bfloat16)

    sums = _sc_swd_kernel(keys2, xp, yn)
    per_sample = sums.reshape(_B, _NPROJ * 16).sum(axis=1)
    return jnp.mean(per_sample / jnp.float32(_L * _D * _NPROJ))


# speedup vs baseline: 1.6204x; 1.0156x over previous
import functools

import jax
import jax.numpy as jnp
from jax import lax
from jax.experimental import pallas as pl
from jax.experimental.pallas import tpu as pltpu
from jax.experimental.pallas import tpu_sc as plsc

_PS = 7
_STRIDE = 2
_NPROJ = 256
_D = 147
_DP = 160
_L = 3721
_LP = 4096
_CHUNK = _LP // 16
_GCH = 128
_NCH = _LP // _GCH
_B = 4
_NTASK = _B * _NPROJ
_NTILE = 32
_TPT = _NTASK // _NTILE


def _patches_t(img):
    p = lax.conv_general_dilated_patches(
        img, filter_shape=(_PS, _PS), window_strides=(_STRIDE, _STRIDE),
        padding="VALID")
    return p.reshape(img.shape[0], _D, _L)


def _proj_tc_kernel(randT_ref, xT_ref, out_ref):
    r = randT_ref[...]
    mu = jnp.mean(r, axis=1, keepdims=True)
    var = jnp.sum((r - mu) ** 2, axis=1, keepdims=True) * (1.0 / (_D - 1))
    rn = r * lax.rsqrt(var)
    x = xT_ref[0]
    acc = lax.dot_general(rn, x, (((1,), (0,)), ((), ())),
                          preferred_element_type=jnp.float32)
    b = lax.bitcast_convert_type(acc, jnp.int32)
    mono = jnp.where(acc < 0, ~b, b | jnp.int32(-2147483648))
    col = lax.broadcasted_iota(jnp.int32, mono.shape, 1)
    mono24 = lax.shift_right_logical(mono, 8)
    out_ref[0] = jnp.where(col >= _L, jnp.int32(0xFFFFFF), mono24)


_sc_mesh = plsc.VectorSubcoreMesh(core_axis_name="c", subcore_axis_name="s")


@functools.partial(
    pl.kernel,
    mesh=_sc_mesh,
    compiler_params=pltpu.CompilerParams(
        needs_layout_passes=False, use_tc_tiling_on_sc=False),
    out_type=jax.ShapeDtypeStruct((_NTASK * 16,), jnp.float32),
    scratch_types=[
        pltpu.VMEM((_LP,), jnp.int32),
        pltpu.VMEM((_LP,), jnp.int32),
        pltpu.VMEM((_LP,), jnp.int32),
        pltpu.VMEM((_LP,), jnp.int32),
        pltpu.VMEM((_LP,), jnp.int32),
        pltpu.VMEM((_LP,), jnp.int32),
        pltpu.VMEM((_LP,), jnp.int32),
        pltpu.VMEM((_LP,), jnp.int32),
        pltpu.VMEM((_LP,), jnp.int32),
        pltpu.VMEM((_LP,), jnp.int32),
        pltpu.VMEM((4096,), jnp.int32),
        pltpu.VMEM((4096,), jnp.int32),
        pltpu.SMEM((256,), jnp.int32),
        pltpu.SMEM((256,), jnp.int32),
        pltpu.VMEM((_GCH, _DP), jnp.bfloat16),
        pltpu.VMEM((_GCH, _DP), jnp.bfloat16),
        pltpu.VMEM((_GCH, _DP), jnp.bfloat16),
        pltpu.VMEM((_GCH, _DP), jnp.bfloat16),
        pltpu.VMEM((_TPT * 16,), jnp.float32),
        pltpu.SemaphoreType.DMA,
        pltpu.SemaphoreType.DMA,
        pltpu.SemaphoreType.DMA,
        pltpu.SemaphoreType.DMA,
        pltpu.SemaphoreType.DMA,
        pltpu.SemaphoreType.DMA,
        pltpu.SemaphoreType.DMA,
        pltpu.SemaphoreType.DMA,
    ],
)
def _sc_swd_kernel(keys_hbm, xp_hbm, yn_hbm, out_hbm,
                   kAx, kBx, pAx, pBx, kAy, kBy, pAy, pBy,
                   idxX, idxY, histx, histy, smx, smy,
                   b0, b1, b2, b3, res,
                   sx0, sx1, sx2, sx3, sy0, sy1, sy2, sy3):
    wid = lax.axis_index("s") * 2 + lax.axis_index("c")
    lanes = lax.iota(jnp.int32, 16)
    ones = jnp.ones((16,), jnp.int32)
    zeros16 = jnp.zeros((16,), jnp.int32)
    gb0 = lanes * _CHUNK

    def radix_pass(xio, yio, shift, first, last, pbase):
        kinx, pinx, koutx, poutx = xio
        kiny, piny, kouty, pouty = yio

        def zb(i, c):
            histx[pl.ds(i * 16, 16)] = zeros16
            histy[pl.ds(i * 16, 16)] = zeros16
            return c
        lax.fori_loop(0, 256, zb, 0, unroll=4)

        _G = 4

        def load_keys(t):
            if first:
                return (plsc.load_gather(kinx, [gb0 + t]),
                        plsc.load_gather(kiny, [gb0 + t]))
            return (kinx[pl.ds(t * 16, 16)], kiny[pl.ds(t * 16, 16)])

        def ph1(i, c):
            t0 = i * 8
            ks = [load_keys(t0 + u) for u in range(8)]
            for kx, ky in ks:
                dgx = lax.shift_right_logical(kx, shift) & 255
                dgy = lax.shift_right_logical(ky, shift) & 255
                plsc.addupdate_scatter(histx, [dgx * 16 + lanes], ones)
                plsc.addupdate_scatter(histy, [dgy * 16 + lanes], ones)
            return c
        lax.fori_loop(0, _CHUNK // 8, ph1, 0)

        def p2a(i, c):
            smx[i] = jnp.sum(histx[pl.ds(i * 16, 16)])
            smy[i] = jnp.sum(histy[pl.ds(i * 16, 16)])
            return c
        lax.fori_loop(0, 256, p2a, 0, unroll=4)

        def p2b(i, carry):
            cx, cy = carry
            tx = smx[i]
            ty = smy[i]
            smx[i] = cx
            smy[i] = cy
            return (cx + tx, cy + ty)
        lax.fori_loop(0, 256, p2b, (jnp.int32(0), jnp.int32(0)))

        def p2c(i, c):
            vx = histx[pl.ds(i * 16, 16)]
            vy = histy[pl.ds(i * 16, 16)]
            histx[pl.ds(i * 16, 16)] = plsc.cumsum(vx) - vx + smx[i]
            histy[pl.ds(i * 16, 16)] = plsc.cumsum(vy) - vy + smy[i]
            return c
        lax.fori_loop(0, 256, p2c, 0, unroll=4)

        def swz(off):
            return lax.shift_left(off & 255, 4) | lax.shift_right_logical(
                off, 8)

        def ph3(i, c):
            t0 = i * _G
            ks = [load_keys(t0 + u) for u in range(_G)]
            if first:
                ps = [(gb0 + t0 + u + pbase,) * 2 for u in range(_G)]
            else:
                ps = [(pinx[pl.ds((t0 + u) * 16, 16)],
                       piny[pl.ds((t0 + u) * 16, 16)]) for u in range(_G)]
            ax = [None] * _G
            ay = [None] * _G
            for u, (kx, ky) in enumerate(ks):
                ax[u] = ((lax.shift_right_logical(kx, shift) & 255) * 16
                         + lanes)
                ay[u] = ((lax.shift_right_logical(ky, shift) & 255) * 16
                         + lanes)
            ox = [plsc.load_gather(histx, [a]) for a in ax]
            oy = [plsc.load_gather(histy, [a]) for a in ay]
            for u in range(1, _G):
                cx = (ax[u] == ax[0]).astype(jnp.int32)
                cy = (ay[u] == ay[0]).astype(jnp.int32)
                for v in range(1, u):
                    cx = cx + (ax[u] == ax[v]).astype(jnp.int32)
                    cy = cy + (ay[u] == ay[v]).astype(jnp.int32)
                ox[u] = ox[u] + cx
                oy[u] = oy[u] + cy
            for u, (kx, ky) in enumerate(ks):
                px, py = ps[u]
                if last:
                    plsc.store_scatter(poutx, [ox[u]], px)
                    plsc.store_scatter(pouty, [oy[u]], py)
                else:
                    sx_ = swz(ox[u])
                    sy_ = swz(oy[u])
                    plsc.store_scatter(koutx, [sx_], kx)
                    plsc.store_scatter(kouty, [sy_], ky)
                    plsc.store_scatter(poutx, [sx_], px)
                    plsc.store_scatter(pouty, [sy_], py)
            for u in range(_G):
                plsc.addupdate_scatter(histx, [ax[u]], ones)
                plsc.addupdate_scatter(histy, [ay[u]], ones)
            return c
        lax.fori_loop(0, _CHUNK // _G, ph3, 0)

    def sort_both(rowx, rowy, pbase):
        pltpu.sync_copy(keys_hbm.at[rowx], kAx)
        pltpu.sync_copy(keys_hbm.at[rowy], kAy)
        radix_pass((kAx, None, kBx, pBx), (kAy, None, kBy, pBy),
                   0, True, False, pbase)
        radix_pass((kBx, pBx, kAx, pAx), (kBy, pBy, kAy, pAy),
                   8, False, False, pbase)
        radix_pass((kAx, pAx, kBx, idxX), (kAy, pAy, kBy, idxY),
                   16, False, True, pbase)

    def task_body(r, c):
        task = wid * _TPT + r
        s = task // _NPROJ
        j = task - s * _NPROJ
        pbase = s * _LP
        sort_both((s * 2) * _NPROJ + j, (s * 2 + 1) * _NPROJ + j, pbase)

        bufs = ((b0, sx0, sy0), (b1, sx1, sy1), (b2, sx2, sy2),
                (b3, sx3, sy3))

        def issue_x(ci, slot):
            buf, sx, _ = bufs[slot]
            pltpu.async_copy(xp_hbm.at[idxX.at[pl.ds(ci * _GCH, _GCH)]],
                             buf, sx)

        def issue_yadd(ci, slot):
            buf, _, sy = bufs[slot]
            pltpu.async_copy(yn_hbm.at[idxY.at[pl.ds(ci * _GCH, _GCH)]],
                             buf, sy, add=True)

        def wait_x(slot):
            buf, sx, _ = bufs[slot]
            pltpu.make_async_copy(xp_hbm.at[pl.ds(0, _GCH)], buf, sx).wait()

        def wait_y(slot):
            buf, _, sy = bufs[slot]
            pltpu.make_async_copy(xp_hbm.at[pl.ds(0, _GCH)], buf, sy).wait()

        issue_x(0, 0)
        issue_x(1, 1)
        wait_x(0)
        issue_yadd(0, 0)

        def chunk_quad(i, accs):
            for b in (0, 1, 2, 3):
                c = 4 * i + b
                m1 = (b + 1) % 4

                @pl.when(c + 1 < _NCH)
                def _():
                    wait_x(m1)
                    issue_yadd(c + 1, m1)

                buf = bufs[b][0]
                wait_y(b)

                def rowloop(ri, carry):
                    a0, a1, a2, a3 = carry
                    r0 = ri * 2
                    ds = [buf[r0 + (q // 5), pl.ds((q % 5) * 32, 32)]
                          for q in range(10)]
                    e = [jnp.abs(d) for d in ds]
                    rs0 = ((e[0] + e[1]) + (e[2] + e[3])) + e[4]
                    rs1 = ((e[5] + e[6]) + (e[7] + e[8])) + e[9]
                    lo0, hi0 = plsc.unpack(
                        rs0, format=plsc.PackFormat.INTERLEAVED)
                    lo1, hi1 = plsc.unpack(
                        rs1, format=plsc.PackFormat.INTERLEAVED)
                    return (a0 + lo0, a1 + hi0, a2 + lo1, a3 + hi1)
                accs = lax.fori_loop(0, _GCH // 2, rowloop, accs, unroll=2)

                @pl.when(c + 2 < _NCH)
                def _():
                    issue_x(c + 2, (b + 2) % 4)
            return accs

        zf = jnp.zeros((16,), jnp.float32)
        accs = lax.fori_loop(0, _NCH // 4, chunk_quad, (zf, zf, zf, zf))
        acc = (accs[0] + accs[1]) + (accs[2] + accs[3])
        res[pl.ds(r * 16, 16)] = acc
        return c

    lax.fori_loop(0, _TPT, task_body, 0)
    pltpu.sync_copy(res, out_hbm.at[pl.ds(wid * _TPT * 16, _TPT * 16)])


def kernel(x, y, rand):
    xT = _patches_t(x)
    yT = _patches_t(y)
    xTp = jnp.pad(xT, ((0, 0), (0, 0), (0, _LP - _L)))
    yTp = jnp.pad(yT, ((0, 0), (0, 0), (0, _LP - _L)))
    xyT = jnp.stack([xTp, yTp], axis=1).reshape(2 * _B, _D, _LP)
    randT = jnp.transpose(rand)

    keys = pl.pallas_call(
        _proj_tc_kernel,
        grid=(2 * _B,),
        in_specs=[
            pl.BlockSpec((_NPROJ, _D), lambda i: (0, 0)),
            pl.BlockSpec((1, _D, _LP), lambda i: (i, 0, 0)),
        ],
        out_specs=pl.BlockSpec((1, _NPROJ, _LP), lambda i: (i, 0, 0)),
        out_shape=jax.ShapeDtypeStruct((2 * _B, _NPROJ, _LP), jnp.int32),
    )(randT, xyT)
    keys2 = keys.reshape(2 * _B * _NPROJ, _LP)

    xp = jnp.pad(jnp.transpose(xT, (0, 2, 1)),
                 ((0, 0), (0, _LP - _L), (0, _DP - _D))
                 ).reshape(_B * _LP, _DP).astype(jnp.bfloat16)
    yn = jnp.pad(jnp.transpose(-yT, (0, 2, 1)),
                 ((0, 0), (0, _LP - _L), (0, _DP - _D))
                 ).reshape(_B * _LP, _DP).astype(jnp.bfloat16)

    sums = _sc_swd_kernel(keys2, xp, yn)
    per_sample = sums.reshape(_B, _NPROJ * 16).sum(axis=1)
    return jnp.mean(per_sample / jnp.float32(_L * _D * _NPROJ))
